# Initial kernel scaffold; baseline (speedup 1.0000x reference)
import jax
import jax.numpy as jnp
from jax.experimental import pallas as pl


def kernel(x, minutes, table):
    raise NotImplementedError("write your pallas kernel here")


# speedup vs baseline: 1.1628x; 1.1628x over previous
import functools

import jax
import jax.numpy as jnp
from jax.experimental import pallas as pl
from jax.experimental.pallas import tpu as pltpu
from jax.experimental.pallas import tpu_sc as plsc

_B, _S, _D = 4096, 200, 64
_N = _B * _S
_W = 128
_STEPS = _N // _W
_NC, _NS = 2, 16
_NW = _NC * _NS
_STEPS_PER_W = _STEPS // _NW
_L = 16


def _sc_embed_add(xf, mf, table):
    mesh = plsc.VectorSubcoreMesh(core_axis_name="core",
                                  subcore_axis_name="subcore")

    @functools.partial(
        pl.kernel,
        out_type=jax.ShapeDtypeStruct((_N, _D), jnp.float32),
        mesh=mesh,
        compiler_params=pltpu.CompilerParams(use_tc_tiling_on_sc=False),
    )
    def k(x_hbm, i_hbm, t_hbm, o_hbm):
        def body(i_vmem, x_vmem, o_vmem):
            pltpu.sync_copy(t_hbm.at[i_vmem.at[0]], o_vmem)

            @pl.loop(0, _W)
            def _(r):
                for c in range(_D // _L):
                    slc = (pl.ds(r, 1), pl.ds(c * _L, _L))
                    o_vmem.at[slc][...] += x_vmem.at[slc][...]

        pltpu.emit_pipeline(
            body,
            grid=(_NC, _NS, _STEPS_PER_W),
            in_specs=[
                pl.BlockSpec(
                    (1, _W),
                    index_map=lambda i, j, k_: (
                        0, (i * _NS + j) * _STEPS_PER_W + k_),
                ),
                pl.BlockSpec(
                    (_W, _D),
                    index_map=lambda i, j, k_: (
                        (i * _NS + j) * _STEPS_PER_W + k_, 0),
                ),
            ],
            out_specs=[
                pl.BlockSpec(
                    (_W, _D),
                    index_map=lambda i, j, k_: (
                        (i * _NS + j) * _STEPS_PER_W + k_, 0),
                ),
            ],
            core_axis_name=("core", "subcore"),
            dimension_semantics=(pltpu.PARALLEL, pltpu.PARALLEL,
                                 pltpu.ARBITRARY),
        )(i_hbm, x_hbm, o_hbm)

    return k(xf, mf, table)


@jax.jit
def kernel(x, minutes, table):
    xf = x.reshape(_N, _D)
    mf = minutes.astype(jnp.int32).reshape(1, _N)
    out = _sc_embed_add(xf, mf, table)
    return out.reshape(_B, _S, _D)
